# baseline (speedup 1.0000x reference)
import functools
import jax
import jax.numpy as jnp
from jax import lax
from jax.experimental import pallas as pl
from jax.experimental.pallas import tpu as pltpu
from jax.experimental.pallas import tpu_sc as plsc

NUM_NODES_K = 50000
EMBED_DIM_K = 128
BATCH_K = 8
CHUNK_N = 400
N_CHUNKS_TOT = NUM_NODES_K // CHUNK_N
N_WORKERS = 32
CHUNKS_PER_W = -(-N_CHUNKS_TOT // N_WORKERS)
NBUF = 2


def _sc_body(t_hbm, o_hbm, b0, b1, i0, i1, o0, o1):
    bufs = [b0, b1]
    isems = [i0, i1]
    osems = [o0, o1]
    wid = lax.axis_index("c") * 16 + lax.axis_index("s")

    def rows_of(i):
        cid = wid + i * N_WORKERS
        return cid, pl.ds(cid * CHUNK_N, CHUNK_N)

    def start_load(i):
        if i >= CHUNKS_PER_W:
            return
        cid, rows = rows_of(i)

        @pl.when(cid < N_CHUNKS_TOT)
        def _():
            pltpu.async_copy(t_hbm.at[rows, :], bufs[i % NBUF],
                             isems[i % NBUF])

    def wait_load(i):
        cid, rows = rows_of(i)

        @pl.when(cid < N_CHUNKS_TOT)
        def _():
            pltpu.make_async_copy(t_hbm.at[rows, :], bufs[i % NBUF],
                                  isems[i % NBUF]).wait()

    def start_stores(i):
        cid, rows = rows_of(i)

        @pl.when(cid < N_CHUNKS_TOT)
        def _():
            for j in range(BATCH_K):
                b = lax.rem(wid + j, BATCH_K)
                pltpu.async_copy(bufs[i % NBUF], o_hbm.at[b, rows, :],
                                 osems[i % NBUF])

    def wait_stores(i):
        if i < 0 or i >= CHUNKS_PER_W:
            return
        cid, rows = rows_of(i)

        @pl.when(cid < N_CHUNKS_TOT)
        def _():
            for b in range(BATCH_K):
                pltpu.make_async_copy(bufs[i % NBUF], o_hbm.at[b, rows, :],
                                      osems[i % NBUF]).wait()

    for i in range(NBUF - 1):
        start_load(i)
    for i in range(CHUNKS_PER_W):
        wait_load(i)
        start_stores(i)
        wait_stores(i - 1)
        start_load(i + NBUF - 1)
    wait_stores(CHUNKS_PER_W - 1)


def kernel(table, batch_size):
    del batch_size
    mesh = plsc.VectorSubcoreMesh(core_axis_name="c", subcore_axis_name="s")
    run = functools.partial(
        pl.kernel,
        mesh=mesh,
        out_type=jax.ShapeDtypeStruct((BATCH_K, NUM_NODES_K, EMBED_DIM_K),
                                      jnp.float32),
        scratch_types=(
            [pltpu.VMEM((CHUNK_N, EMBED_DIM_K), jnp.float32)] * NBUF
            + [pltpu.SemaphoreType.DMA] * (2 * NBUF)
        ),
    )(_sc_body)
    return run(table)


# speedup vs baseline: 1.0030x; 1.0030x over previous
import functools
import jax
import jax.numpy as jnp
from jax import lax
from jax.experimental import pallas as pl
from jax.experimental.pallas import tpu as pltpu
from jax.experimental.pallas import tpu_sc as plsc

NUM_NODES_K = 50000
EMBED_DIM_K = 128
BATCH_K = 8
CHUNK_N = 400
N_CHUNKS_TOT = NUM_NODES_K // CHUNK_N
N_WORKERS = 32
CHUNKS_PER_W = -(-N_CHUNKS_TOT // N_WORKERS)
NBUF = 2


def _sc_body(t_hbm, o_hbm, b0, b1, i0, i1, o0, o1):
    bufs = [b0, b1]
    isems = [i0, i1]
    osems = [o0, o1]
    wid = lax.axis_index("c") * 16 + lax.axis_index("s")

    def rows_of(i):
        cid = wid + i * N_WORKERS
        return cid, pl.ds(cid * CHUNK_N, CHUNK_N)

    def start_load(i):
        if i >= CHUNKS_PER_W:
            return
        cid, rows = rows_of(i)

        @pl.when(cid < N_CHUNKS_TOT)
        def _():
            pltpu.async_copy(t_hbm.at[rows, :], bufs[i % NBUF],
                             isems[i % NBUF])

    def wait_load(i):
        cid, rows = rows_of(i)

        @pl.when(cid < N_CHUNKS_TOT)
        def _():
            pltpu.make_async_copy(t_hbm.at[rows, :], bufs[i % NBUF],
                                  isems[i % NBUF]).wait()

    def start_stores(i):
        cid, rows = rows_of(i)

        @pl.when(cid < N_CHUNKS_TOT)
        def _():
            for b in range(BATCH_K):
                pltpu.async_copy(bufs[i % NBUF], o_hbm.at[b, rows, :],
                                 osems[i % NBUF])

    def wait_stores(i):
        if i < 0 or i >= CHUNKS_PER_W:
            return
        cid, rows = rows_of(i)

        @pl.when(cid < N_CHUNKS_TOT)
        def _():
            for b in range(BATCH_K):
                pltpu.make_async_copy(bufs[i % NBUF], o_hbm.at[b, rows, :],
                                      osems[i % NBUF]).wait()

    for i in range(NBUF - 1):
        start_load(i)
    for i in range(CHUNKS_PER_W):
        wait_load(i)
        start_stores(i)
        wait_stores(i - 1)
        start_load(i + NBUF - 1)
    wait_stores(CHUNKS_PER_W - 1)


def kernel(table, batch_size):
    del batch_size
    mesh = plsc.VectorSubcoreMesh(core_axis_name="c", subcore_axis_name="s")
    run = functools.partial(
        pl.kernel,
        mesh=mesh,
        out_type=jax.ShapeDtypeStruct((BATCH_K, NUM_NODES_K, EMBED_DIM_K),
                                      jnp.float32),
        scratch_types=(
            [pltpu.VMEM((CHUNK_N, EMBED_DIM_K), jnp.float32)] * NBUF
            + [pltpu.SemaphoreType.DMA] * (2 * NBUF)
        ),
    )(_sc_body)
    return run(table)
